# baseline (speedup 1.0000x reference)
import functools

import jax
import jax.numpy as jnp
from jax import lax
from jax.experimental import pallas as pl
from jax.experimental.pallas import tpu as pltpu
from jax.experimental.pallas import tpu_sc as plsc

R = 16384
C = 4096
K = 3
KP = 4
NW = 32
RPW = R // NW
BPW = RPW // 8
L = 16
COLS = (1, 3, 0)
OPW = RPW // 128 * KP * 128


@functools.partial(
    pl.kernel,
    out_type=jax.ShapeDtypeStruct((KP * R,), jnp.float32),
    mesh=plsc.VectorSubcoreMesh(core_axis_name="c", subcore_axis_name="s"),
    scratch_types=[
        pltpu.VMEM((BPW, 8, 128), jnp.float32),
        pltpu.VMEM((OPW,), jnp.float32),
    ],
    compiler_params=pltpu.CompilerParams(
        use_tc_tiling_on_sc=True,
        needs_layout_passes=False,
        skip_device_barrier=True,
    ),
)
def _sc_gather(x_hbm, out_hbm, blk_v, out_v):
    wid = lax.axis_index("s") * 2 + lax.axis_index("c")

    pltpu.sync_copy(
        x_hbm.at[pl.ds(wid * BPW, BPW), :, pl.ds(0, 16)],
        blk_v.at[:, :, pl.ds(0, 16)],
    )

    iota = lax.iota(jnp.int32, L)

    for k in range(RPW // L):
        m = k * L + iota
        b = lax.shift_right_logical(m, 3)
        r8 = lax.bitwise_and(m, 7)
        for jj, j in enumerate(COLS):
            lane = jnp.full((L,), j, jnp.int32)
            vals = plsc.load_gather(blk_v, [b, r8, lane])
            off = (k // 8) * (KP * 128) + jj * 128 + (k % 8) * L
            out_v[pl.ds(off, L)] = vals

    pltpu.sync_copy(out_v, out_hbm.at[pl.ds(wid * OPW, OPW)])


def kernel(x):
    x3 = x.reshape(R // 8, 8, C)
    out_flat = _sc_gather(x3)
    out = out_flat.reshape(R // 128, KP, 128).transpose(0, 2, 1)
    return out.reshape(R, KP)[:, :K]


# speedup vs baseline: 1.0277x; 1.0277x over previous
import functools

import jax
import jax.numpy as jnp
from jax import lax
from jax.experimental import pallas as pl
from jax.experimental.pallas import tpu as pltpu
from jax.experimental.pallas import tpu_sc as plsc

R = 16384
C = 4096
K = 3
KP = 4
NW = 32
RPW = R // NW
BPW = RPW // 8
L = 16
COLS = (1, 3, 0)
OPW = RPW // 128 * KP * 128


@functools.partial(
    pl.kernel,
    out_type=jax.ShapeDtypeStruct((KP * R,), jnp.float32),
    mesh=plsc.VectorSubcoreMesh(core_axis_name="c", subcore_axis_name="s"),
    scratch_types=[
        pltpu.VMEM((BPW, 8, 128), jnp.float32),
        pltpu.VMEM((OPW,), jnp.float32),
    ],
    compiler_params=pltpu.CompilerParams(
        use_tc_tiling_on_sc=True,
        needs_layout_passes=False,
        skip_device_barrier=True,
    ),
)
def _sc_gather(x_hbm, out_hbm, blk_v, out_v):
    wid = lax.axis_index("s") * 2 + lax.axis_index("c")

    pltpu.sync_copy(
        x_hbm.at[pl.ds(wid * BPW, BPW), :, pl.ds(0, 16)],
        blk_v.at[:, :, pl.ds(0, 16)],
    )

    iota = lax.iota(jnp.int32, L)

    def extract(k, carry):
        m = k * L + iota
        b = lax.shift_right_logical(m, 3)
        r8 = lax.bitwise_and(m, 7)
        for jj, j in enumerate(COLS):
            lane = jnp.full((L,), j, jnp.int32)
            vals = plsc.load_gather(blk_v, [b, r8, lane])
            off = (k // 8) * (KP * 128) + jj * 128 + (k % 8) * L
            out_v[pl.ds(off, L)] = vals
        return carry

    lax.fori_loop(0, RPW // L, extract, None)

    pltpu.sync_copy(out_v, out_hbm.at[pl.ds(wid * OPW, OPW)])


def kernel(x):
    x3 = x.reshape(R // 8, 8, C)
    out_flat = _sc_gather(x3)
    out = out_flat.reshape(R // 128, KP, 128).transpose(0, 2, 1)
    return out.reshape(R, KP)[:, :K]
